# baseline (speedup 1.0000x reference)
import functools

import jax
import jax.numpy as jnp
from jax import lax
from jax.experimental import pallas as pl
from jax.experimental.pallas import tpu as pltpu
from jax.experimental.pallas import tpu_sc as plsc

T, B, D = 2048, 16, 1024
C = 32
NCHUNK = T // C
NSUB = 16
NCORE = 2
BPC = B // NCORE
LANES = 16
HALF = D // 2
NEG = float(jnp.finfo(jnp.float32).min)
IMIN = -(2**31)


def _nli_max_body(x_hbm, len_hbm, out_hbm, buf, acc, len_v, cmb, obuf, partial):
    c = lax.axis_index("c")
    s = lax.axis_index("s")
    iota16 = lax.iota(jnp.int32, 16)
    negvec = jnp.full((LANES,), NEG, dtype=jnp.float32)

    pltpu.sync_copy(len_hbm, len_v.at[pl.ds(0, LANES)])

    def extract_i32(idx):
        v = len_v[pl.ds(idx, LANES)]
        return v[0]

    def init_body(i, carry):
        acc[pl.ds(i * LANES, LANES)] = negvec
        return carry

    lax.fori_loop(0, BPC * D // LANES, init_body, 0)

    def batch_body(bl, carry):
        b = c * BPC + bl
        lenb = extract_i32(b)
        k0 = lax.rem(s + 3 * bl, NSUB)
        accbase = bl * D

        def m_body(m, mcarry):
            t0 = (k0 + m * NSUB) * C

            @pl.when(t0 < lenb)
            def _():
                pltpu.sync_copy(x_hbm.at[pl.ds(t0, C), pl.ds(b, 1)], buf)
                nvalid = lenb - t0

                def j_body(j, jcarry):
                    col = j * LANES
                    a = acc[pl.ds(accbase + col, LANES)]
                    for r in range(C):
                        xv = buf[r, 0, pl.ds(col, LANES)]
                        xv = jnp.where(r < nvalid, xv, negvec)
                        a = jnp.maximum(a, xv)
                    acc[pl.ds(accbase + col, LANES)] = a
                    return jcarry

                lax.fori_loop(0, D // LANES, j_body, 0)

            return mcarry

        lax.fori_loop(0, NCHUNK // NSUB, m_body, 0)
        return carry

    lax.fori_loop(0, BPC, batch_body, 0)

    pltpu.sync_copy(acc, partial.at[s])
    plsc.subcore_barrier()

    bl2 = s // 2
    h = s % 2
    col0 = bl2 * D + h * HALF
    pltpu.sync_copy(partial.at[:, pl.ds(col0, HALF)], cmb)

    def cmb_body(jj, carry):
        colc = jj * LANES
        a = cmb[0, pl.ds(colc, LANES)]
        for row in range(1, NSUB):
            a = jnp.maximum(a, cmb[row, pl.ds(colc, LANES)])
        obuf[pl.ds(colc, LANES)] = a
        return carry

    lax.fori_loop(0, HALF // LANES, cmb_body, 0)

    bout = c * BPC + bl2
    pltpu.sync_copy(obuf, out_hbm.at[bout, pl.ds(h * HALF, HALF)])


def kernel(inputs, lengths):
    mesh = plsc.VectorSubcoreMesh(core_axis_name="c", subcore_axis_name="s")
    f = functools.partial(
        pl.kernel,
        mesh=mesh,
        out_type=jax.ShapeDtypeStruct((B, D), jnp.float32),
        scratch_types=[
            pltpu.VMEM((C, 1, D), jnp.float32),
            pltpu.VMEM((BPC * D,), jnp.float32),
            pltpu.VMEM((2 * LANES,), jnp.int32),
            pltpu.VMEM((NSUB, HALF), jnp.float32),
            pltpu.VMEM((HALF,), jnp.float32),
            pltpu.VMEM_SHARED((NSUB, BPC * D), jnp.float32),
        ],
    )(_nli_max_body)
    return f(inputs, lengths)
